# Initial kernel scaffold; baseline (speedup 1.0000x reference)
import jax
import jax.numpy as jnp
from jax.experimental import pallas as pl


def kernel(x, adj_indices, adj_values, W, b, prelu_alpha):
    raise NotImplementedError("write your pallas kernel here")


# speedup vs baseline: 3.0563x; 3.0563x over previous
import functools

import jax
import jax.numpy as jnp
from jax import lax
from jax.experimental import pallas as pl
from jax.experimental.pallas import tpu as pltpu
from jax.experimental.pallas import tpu_sc as plsc

N = 10000
E = 320000
D = 128

NC = 2
NS = 16
NW = NC * NS

C = 128
KCHUNKS = 80
EW = KCHUNKS * C
E_PAD = EW * NW
RT = 624
RT_LAST = N - 15 * RT


def _matmul_body(x_ref, wt_ref, b_ref, o_ref):
    o_ref[...] = (
        jnp.dot(x_ref[...], wt_ref[...], preferred_element_type=jnp.float32)
        + b_ref[...]
    )


def _linear(x, W, b):
    blk = 1000
    grid = N // blk
    return pl.pallas_call(
        _matmul_body,
        grid=(grid,),
        in_specs=[
            pl.BlockSpec((blk, D), lambda i: (i, 0)),
            pl.BlockSpec((D, D), lambda i: (0, 0)),
            pl.BlockSpec((1, D), lambda i: (0, 0)),
        ],
        out_specs=pl.BlockSpec((blk, D), lambda i: (i, 0)),
        out_shape=jax.ShapeDtypeStruct((N, D), jnp.float32),
    )(x, W.T, b.reshape(1, D))


def _spmm_body(h, col2, row2, val2, zeros, p0, p1,
               col_all, row_all, val_all, rows, agg, sem):
    cid = lax.axis_index("c")
    sid = lax.axis_index("s")
    w = cid * NS + sid
    r0 = pl.multiple_of(sid * RT, 8)

    @pl.when(sid < 15)
    def _():
        pltpu.sync_copy(zeros.at[pl.ds(r0, RT), :],
                        agg.at[pl.ds(r0, RT), :])

    @pl.when(sid == 15)
    def _():
        pltpu.sync_copy(zeros.at[pl.ds(r0, RT_LAST), :],
                        agg.at[pl.ds(r0, RT_LAST), :])

    pltpu.sync_copy(col2.at[w], col_all)
    pltpu.sync_copy(row2.at[w], row_all)
    pltpu.sync_copy(val2.at[w], val_all)

    plsc.subcore_barrier()

    def chunk(k, carry):
        pltpu.async_copy(h.at[col_all.at[k]], rows, sem).wait()

        def scale(g, c2):
            v16 = val_all[k, pl.ds(g * 16, 16)]
            for l in range(16):
                v = v16[l]
                i = g * 16 + l
                for j in range(D // 16):
                    sl = pl.ds(j * 16, 16)
                    rows[i, sl] = rows[i, sl] * v
            return c2
        lax.fori_loop(0, C // 16, scale, 0)

        pltpu.sync_copy(rows, agg.at[row_all.at[k]], add=True)
        return carry

    lax.fori_loop(0, KCHUNKS, chunk, 0)

    plsc.subcore_barrier()

    @pl.when((cid == 0) & (sid < 15))
    def _():
        pltpu.sync_copy(agg.at[pl.ds(r0, RT), :], p0.at[pl.ds(r0, RT), :])

    @pl.when((cid == 0) & (sid == 15))
    def _():
        pltpu.sync_copy(agg.at[pl.ds(r0, RT_LAST), :],
                        p0.at[pl.ds(r0, RT_LAST), :])

    @pl.when((cid == 1) & (sid < 15))
    def _():
        pltpu.sync_copy(agg.at[pl.ds(r0, RT), :], p1.at[pl.ds(r0, RT), :])

    @pl.when((cid == 1) & (sid == 15))
    def _():
        pltpu.sync_copy(agg.at[pl.ds(r0, RT_LAST), :],
                        p1.at[pl.ds(r0, RT_LAST), :])


def _spmm(h, col2, row2, val2, zeros):
    mesh = plsc.VectorSubcoreMesh(
        core_axis_name="c", subcore_axis_name="s",
        num_cores=NC, num_subcores=NS)
    f = functools.partial(
        pl.kernel,
        out_type=[
            jax.ShapeDtypeStruct((N, D), jnp.float32),
            jax.ShapeDtypeStruct((N, D), jnp.float32),
        ],
        mesh=mesh,
        scratch_types=[
            pltpu.VMEM((KCHUNKS, C), jnp.int32),
            pltpu.VMEM((KCHUNKS, C), jnp.int32),
            pltpu.VMEM((KCHUNKS, C), jnp.float32),
            pltpu.VMEM((C, D), jnp.float32),
            pltpu.VMEM_SHARED((N, D), jnp.float32),
            pltpu.SemaphoreType.DMA,
        ],
    )(_spmm_body)
    return f(h, col2, row2, val2, zeros)


def _combine_body(p0_ref, p1_ref, alpha_ref, o_ref):
    s = p0_ref[...] + p1_ref[...]
    a = alpha_ref[0]
    o_ref[...] = jnp.maximum(s, 0.0) + a * jnp.minimum(s, 0.0)


def _combine(p0, p1, alpha):
    blk = 1000
    grid = N // blk
    return pl.pallas_call(
        _combine_body,
        grid=(grid,),
        in_specs=[
            pl.BlockSpec((blk, D), lambda i: (i, 0)),
            pl.BlockSpec((blk, D), lambda i: (i, 0)),
            pl.BlockSpec(memory_space=pltpu.SMEM),
        ],
        out_specs=pl.BlockSpec((blk, D), lambda i: (i, 0)),
        out_shape=jax.ShapeDtypeStruct((N, D), jnp.float32),
    )(p0, p1, alpha)


@jax.jit
def kernel(x, adj_indices, adj_values, W, b, prelu_alpha):
    h = _linear(x, W, b)
    pad = E_PAD - E
    col2 = jnp.pad(adj_indices[1], (0, pad)).reshape(NW, KCHUNKS, C)
    row2 = jnp.pad(adj_indices[0], (0, pad)).reshape(NW, KCHUNKS, C)
    val2 = jnp.pad(adj_values, (0, pad)).reshape(NW, KCHUNKS, C)
    zeros = jnp.zeros((N, D), jnp.float32)
    p0, p1 = _spmm(h, col2, row2, val2, zeros)
    return _combine(p0, p1, prelu_alpha)


# speedup vs baseline: 3.4330x; 1.1233x over previous
import functools

import jax
import jax.numpy as jnp
from jax import lax
from jax.experimental import pallas as pl
from jax.experimental.pallas import tpu as pltpu
from jax.experimental.pallas import tpu_sc as plsc

N = 10000
E = 320000
D = 128

NC = 2
NS = 16
NW = NC * NS

C = 128
KCHUNKS = 80
EW = KCHUNKS * C
E_PAD = EW * NW
SB = 8
NBLK = KCHUNKS // SB
RT = 624
RT_LAST = N - 15 * RT


def _matmul_body(x_ref, wt_ref, b_ref, o_ref):
    o_ref[...] = (
        jnp.dot(x_ref[...], wt_ref[...], preferred_element_type=jnp.float32)
        + b_ref[...]
    )


def _linear(x, W, b):
    blk = 1000
    grid = N // blk
    return pl.pallas_call(
        _matmul_body,
        grid=(grid,),
        in_specs=[
            pl.BlockSpec((blk, D), lambda i: (i, 0)),
            pl.BlockSpec((D, D), lambda i: (0, 0)),
            pl.BlockSpec((1, D), lambda i: (0, 0)),
        ],
        out_specs=pl.BlockSpec((blk, D), lambda i: (i, 0)),
        out_shape=jax.ShapeDtypeStruct((N, D), jnp.float32),
    )(x, W.T, b.reshape(1, D))


def _spmm_body(h, col2, row2, val2, p0, p1,
               colb, rowb, valb, rows, agg,
               gs0, gs1, ss0, ss1, sts):
    cid = lax.axis_index("c")
    sid = lax.axis_index("s")
    w = cid * NS + sid
    r0 = pl.multiple_of(sid * RT, 8)
    gsem = (gs0, gs1)
    ssem = (ss0, ss1)
    zero16 = jnp.zeros((16,), jnp.float32)

    def zrow(r, c2):
        for f in range(D // 16):
            rows[0, r, pl.ds(f * 16, 16)] = zero16
        return c2
    lax.fori_loop(0, C, zrow, 0)

    @pl.when(sid < 15)
    def _():
        for m in range(4):
            pltpu.sync_copy(rows.at[0],
                            agg.at[pl.ds(r0 + m * 128, 128), :])
        pltpu.sync_copy(rows.at[0, pl.ds(0, 112), :],
                        agg.at[pl.ds(r0 + 512, 112), :])

    @pl.when(sid == 15)
    def _():
        for m in range(5):
            pltpu.sync_copy(rows.at[0],
                            agg.at[pl.ds(r0 + m * 128, 128), :])

    def stage_start(bnext, pst):
        off = pl.multiple_of(bnext * SB, 8)
        pltpu.async_copy(col2.at[w, pl.ds(off, SB), :], colb.at[pst], sts)
        pltpu.async_copy(row2.at[w, pl.ds(off, SB), :], rowb.at[pst], sts)
        pltpu.async_copy(val2.at[w, pl.ds(off, SB), :], valb.at[pst], sts)

    def stage_wait(pst):
        pltpu.make_async_copy(col2.at[w, pl.ds(0, SB), :],
                              colb.at[pst], sts).wait()
        pltpu.make_async_copy(row2.at[w, pl.ds(0, SB), :],
                              rowb.at[pst], sts).wait()
        pltpu.make_async_copy(val2.at[w, pl.ds(0, SB), :],
                              valb.at[pst], sts).wait()

    def gather_start(bp, j2, p):
        pltpu.async_copy(h.at[colb.at[bp, j2]], rows.at[p], gsem[p])

    def gather_wait(p):
        pltpu.make_async_copy(h.at[pl.ds(0, C), :],
                              rows.at[p], gsem[p]).wait()

    def scatter_start(bp, j2, p):
        pltpu.async_copy(rows.at[p], agg.at[rowb.at[bp, j2]],
                         ssem[p], add=True)

    def scatter_wait(p):
        pltpu.make_async_copy(h.at[pl.ds(0, C), :],
                              rows.at[p], ssem[p]).wait()

    def scale(bp, j2, p):
        def grp(g, c2):
            v16 = valb[bp, j2, pl.ds(g * 16, 16)]
            for l in range(16):
                v = v16[l]
                i = g * 16 + l
                for f in range(D // 16):
                    sl = pl.ds(f * 16, 16)
                    rows[p, i, sl] = rows[p, i, sl] * v
            return c2
        lax.fori_loop(0, C // 16, grp, 0)

    stage_start(0, 0)
    stage_wait(0)
    gather_start(0, 0, 0)
    stage_start(1, 1)

    plsc.subcore_barrier()

    def blockpair(i, c1):
        for hh in range(2):
            b = 2 * i + hh
            bp = hh

            def pair(jj, c2):
                if hh == 0:
                    @pl.when((i > 0) | (jj > 0))
                    def _():
                        scatter_wait(1)

                    @pl.when((jj == 0) & (i > 0))
                    def _():
                        stage_start(b + 1, 1 - bp)
                else:
                    scatter_wait(1)

                    @pl.when((jj == 0) & (i < 4))
                    def _():
                        stage_start(b + 1, 1 - bp)
                gather_start(bp, 2 * jj + 1, 1)
                gather_wait(0)
                scale(bp, 2 * jj, 0)
                scatter_start(bp, 2 * jj, 0)

                scatter_wait(0)
                if hh == 0:
                    @pl.when(jj == 3)
                    def _():
                        stage_wait(1 - bp)
                        gather_start(1 - bp, 0, 0)

                    @pl.when(jj < 3)
                    def _():
                        gather_start(bp, 2 * jj + 2, 0)
                else:
                    @pl.when((jj == 3) & (i < 4))
                    def _():
                        stage_wait(1 - bp)
                        gather_start(1 - bp, 0, 0)

                    @pl.when(jj < 3)
                    def _():
                        gather_start(bp, 2 * jj + 2, 0)
                gather_wait(1)
                scale(bp, 2 * jj + 1, 1)
                scatter_start(bp, 2 * jj + 1, 1)
                return c2

            lax.fori_loop(0, SB // 2, pair, c1)
        return c1

    lax.fori_loop(0, NBLK // 2, blockpair, 0)

    scatter_wait(1)
    plsc.subcore_barrier()

    @pl.when((cid == 0) & (sid < 15))
    def _():
        pltpu.sync_copy(agg.at[pl.ds(r0, RT), :], p0.at[pl.ds(r0, RT), :])

    @pl.when((cid == 0) & (sid == 15))
    def _():
        pltpu.sync_copy(agg.at[pl.ds(r0, RT_LAST), :],
                        p0.at[pl.ds(r0, RT_LAST), :])

    @pl.when((cid == 1) & (sid < 15))
    def _():
        pltpu.sync_copy(agg.at[pl.ds(r0, RT), :], p1.at[pl.ds(r0, RT), :])

    @pl.when((cid == 1) & (sid == 15))
    def _():
        pltpu.sync_copy(agg.at[pl.ds(r0, RT_LAST), :],
                        p1.at[pl.ds(r0, RT_LAST), :])


def _spmm(h, col2, row2, val2):
    mesh = plsc.VectorSubcoreMesh(
        core_axis_name="c", subcore_axis_name="s",
        num_cores=NC, num_subcores=NS)
    f = functools.partial(
        pl.kernel,
        out_type=[
            jax.ShapeDtypeStruct((N, D), jnp.float32),
            jax.ShapeDtypeStruct((N, D), jnp.float32),
        ],
        mesh=mesh,
        scratch_types=[
            pltpu.VMEM((2, SB, C), jnp.int32),
            pltpu.VMEM((2, SB, C), jnp.int32),
            pltpu.VMEM((2, SB, C), jnp.float32),
            pltpu.VMEM((2, C, D), jnp.float32),
            pltpu.VMEM_SHARED((N, D), jnp.float32),
            pltpu.SemaphoreType.DMA,
            pltpu.SemaphoreType.DMA,
            pltpu.SemaphoreType.DMA,
            pltpu.SemaphoreType.DMA,
            pltpu.SemaphoreType.DMA,
        ],
    )(_spmm_body)
    return f(h, col2, row2, val2)


def _combine_body(p0_ref, p1_ref, alpha_ref, o_ref):
    s = p0_ref[...] + p1_ref[...]
    a = alpha_ref[0]
    o_ref[...] = jnp.maximum(s, 0.0) + a * jnp.minimum(s, 0.0)


def _combine(p0, p1, alpha):
    blk = 1000
    grid = N // blk
    return pl.pallas_call(
        _combine_body,
        grid=(grid,),
        in_specs=[
            pl.BlockSpec((blk, D), lambda i: (i, 0)),
            pl.BlockSpec((blk, D), lambda i: (i, 0)),
            pl.BlockSpec(memory_space=pltpu.SMEM),
        ],
        out_specs=pl.BlockSpec((blk, D), lambda i: (i, 0)),
        out_shape=jax.ShapeDtypeStruct((N, D), jnp.float32),
    )(p0, p1, alpha)


@jax.jit
def kernel(x, adj_indices, adj_values, W, b, prelu_alpha):
    h = _linear(x, W, b)
    pad = E_PAD - E
    col2 = jnp.pad(adj_indices[1], (0, pad)).reshape(NW, KCHUNKS, C)
    row2 = jnp.pad(adj_indices[0], (0, pad)).reshape(NW, KCHUNKS, C)
    val2 = jnp.pad(adj_values, (0, pad)).reshape(NW, KCHUNKS, C)
    p0, p1 = _spmm(h, col2, row2, val2)
    return _combine(p0, p1, prelu_alpha)
